# Initial kernel scaffold; baseline (speedup 1.0000x reference)
import jax
import jax.numpy as jnp
from jax.experimental import pallas as pl


def kernel(fmap1, fmap2, feat0, conv_w, proj_w, coords):
    raise NotImplementedError("write your pallas kernel here")


# speedup vs baseline: 3.2982x; 3.2982x over previous
import functools

import jax
import jax.numpy as jnp
import numpy as np
from jax import lax
from jax.experimental import pallas as pl
from jax.experimental.pallas import tpu as pltpu
from jax.experimental.pallas import tpu_sc as plsc

G = 8
L = 4
R = 4
H = 64
W = 128
C = 64
COLS = 256
LEV_OFF = (0, 128, 192, 224)
LEV_D = (128, 64, 32, 16)
SLAB = G * 16 * COLS
OSLAB = G * 16 * 2 * L * 9
N_UNITS = H * (W // 16)
INV_SQRT_G = 1.0 / np.sqrt(float(G))


def _avg_matrix():
    P = np.zeros((128, 112), dtype=np.float32)
    col = 0
    for i in range(1, 4):
        d = 128 >> i
        s = 1 << i
        for e in range(d):
            P[e * s:(e + 1) * s, col + e] = 1.0 / s
        col += d
    return jnp.asarray(P)


def _tc_tables_body(f1_ref, f2_ref, feat0_ref, w2_ref, proj_ref, pavg_ref,
                    ftab_ref, gtab_ref):
    h = pl.program_id(0)
    slabs = []
    for ky in range(3):
        hh = h + ky - 1
        hc = jnp.clip(hh, 0, H - 1)
        scale = jnp.where((hh >= 0) & (hh < H), INV_SQRT_G, 0.0).astype(jnp.float32)
        a = f1_ref[hc]
        b = f2_ref[hc]
        for g in range(G):
            cvg = lax.dot_general(a[g], b[g], (((0,), (0,)), ((), ())),
                                  preferred_element_type=jnp.float32)
            slabs.append(cvg * scale)
    cv3 = jnp.stack(slabs)

    Y = lax.dot_general(w2_ref[...], cv3, (((0,), (0,)), ((), ())),
                        preferred_element_type=jnp.float32)
    zrow = jnp.zeros((72, 1, 128), jnp.float32)
    zcol = jnp.zeros((72, 130, 1), jnp.float32)
    Ypad = jnp.concatenate([zrow, Y, zrow], axis=1)
    Ypad = jnp.concatenate([zcol, Ypad, zcol], axis=2)

    pbT = lax.dot_general(feat0_ref[h], proj_ref[...], (((0,), (1,)), ((), ())),
                          preferred_element_type=jnp.float32)

    for g in range(G):
        acc = jnp.broadcast_to(pbT[:, g:g + 1], (128, 128))
        for kx in range(3):
            for kd in range(3):
                j2 = g * 9 + kx * 3 + kd
                acc = acc + Ypad[j2, kx:kx + 128, kd:kd + 128]
        F0 = cv3[8 + g]
        Fr = lax.dot_general(F0, pavg_ref[...], (((1,), (0,)), ((), ())),
                             preferred_element_type=jnp.float32)
        Fall = jnp.concatenate([F0, Fr, jnp.zeros((128, 16), jnp.float32)], axis=1)
        ftab_ref[0, :, g] = Fall.reshape(8, 16, COLS)
        Gr = lax.dot_general(acc, pavg_ref[...], (((1,), (0,)), ((), ())),
                             preferred_element_type=jnp.float32)
        Gall = jnp.concatenate([acc, Gr, jnp.zeros((128, 16), jnp.float32)], axis=1)
        gtab_ref[0, :, g] = Gall.reshape(8, 16, COLS)


def _build_tables(f1r, f2r, feat0r, w2, proj_w, pavg):
    full = lambda s: pl.BlockSpec(s, lambda h: (0,) * len(s))
    return pl.pallas_call(
        _tc_tables_body,
        grid=(H,),
        in_specs=[
            full((H, G, 8, W)),
            full((H, G, 8, W)),
            full((H, C, W)),
            full((24, 72)),
            full((G, C)),
            full((128, 112)),
        ],
        out_specs=[
            pl.BlockSpec((1, 8, G, 16, COLS), lambda h: (h, 0, 0, 0, 0)),
            pl.BlockSpec((1, 8, G, 16, COLS), lambda h: (h, 0, 0, 0, 0)),
        ],
        out_shape=[
            jax.ShapeDtypeStruct((H, 8, G, 16, COLS), jnp.float32),
            jax.ShapeDtypeStruct((H, 8, G, 16, COLS), jnp.float32),
        ],
        compiler_params=pltpu.CompilerParams(
            dimension_semantics=("arbitrary",),
        ),
    )(f1r, f2r, feat0r, w2, proj_w, pavg)


def _sc_sample_body(ftab, gtab, coords_r, out_hbm,
                    fbuf, gbuf, obuf, cbuf, i0buf, i1buf, wbuf):
    nc = 2
    wid = lax.axis_index("s") * nc + lax.axis_index("c")
    units_per = N_UNITS // 32
    lane = lax.iota(jnp.int32, 16)
    lrow = lane * COLS

    def unit_body(j, _):
        u = wid * units_per + j
        pltpu.sync_copy(ftab.at[pl.ds(u * SLAB, SLAB)], fbuf)
        pltpu.sync_copy(gtab.at[pl.ds(u * SLAB, SLAB)], gbuf)
        pltpu.sync_copy(coords_r.at[pl.ds(u * 16, 16)], cbuf)
        c = cbuf[...]
        for i in range(L):
            ci = c * (0.5 ** i)
            di = LEV_D[i]
            off = LEV_OFF[i]
            for k in range(9):
                pos = ci + float(k - R)
                t0 = pos.astype(jnp.int32)
                fl = jnp.where(t0.astype(jnp.float32) > pos, t0 - 1, t0)
                wfrac = pos - fl.astype(jnp.float32)
                x0 = jnp.clip(fl, 0, di - 1) + off
                x1 = jnp.clip(fl + 1, 0, di - 1) + off
                t = i * 9 + k
                i0buf[pl.ds(t * 16, 16)] = x0
                i1buf[pl.ds(t * 16, 16)] = x1
                wbuf[pl.ds(t * 16, 16)] = wfrac

        def g_body(g, _):
            rbase = g * 16 * COLS + lrow
            obase = g * 16 * 72 + lane * 72

            def tap_body(t, _):
                i_lev = t // 9
                col0 = t + 9 * i_lev
                i0 = rbase + i0buf[pl.ds(t * 16, 16)]
                i1 = rbase + i1buf[pl.ds(t * 16, 16)]
                wf = wbuf[pl.ds(t * 16, 16)]
                f0 = plsc.load_gather(fbuf, [i0])
                f1 = plsc.load_gather(fbuf, [i1])
                plsc.store_scatter(obuf, [obase + col0],
                                   f0 * (1.0 - wf) + f1 * wf)
                g0 = plsc.load_gather(gbuf, [i0])
                g1 = plsc.load_gather(gbuf, [i1])
                plsc.store_scatter(obuf, [obase + col0 + 9],
                                   g0 * (1.0 - wf) + g1 * wf)
                return 0

            lax.fori_loop(0, L * 9, tap_body, 0)
            return 0

        lax.fori_loop(0, G, g_body, 0)
        pltpu.sync_copy(obuf, out_hbm.at[pl.ds(u * OSLAB, OSLAB)])
        return 0

    lax.fori_loop(0, units_per, unit_body, 0)


def _sc_sample(ftab_flat, gtab_flat, coords_flat):
    mesh = plsc.VectorSubcoreMesh(core_axis_name="c", subcore_axis_name="s")
    kern = functools.partial(
        pl.kernel,
        out_type=jax.ShapeDtypeStruct((N_UNITS * OSLAB,), jnp.float32),
        mesh=mesh,
        scratch_types=[
            pltpu.VMEM((SLAB,), jnp.float32),
            pltpu.VMEM((SLAB,), jnp.float32),
            pltpu.VMEM((OSLAB,), jnp.float32),
            pltpu.VMEM((16,), jnp.float32),
            pltpu.VMEM((L * 9 * 16,), jnp.int32),
            pltpu.VMEM((L * 9 * 16,), jnp.int32),
            pltpu.VMEM((L * 9 * 16,), jnp.float32),
        ],
        compiler_params=pltpu.CompilerParams(needs_layout_passes=False),
    )(_sc_sample_body)
    return kern(ftab_flat, gtab_flat, coords_flat)


def kernel(fmap1, fmap2, feat0, conv_w, proj_w, coords):
    f1r = fmap1[0].transpose(1, 0, 2).reshape(H, G, 8, W)
    f2r = fmap2[0].transpose(1, 0, 2).reshape(H, G, 8, W)
    feat0r = feat0[0].transpose(1, 0, 2)
    w2 = conv_w.transpose(3, 1, 0, 4, 2).reshape(24, 72)
    pavg = _avg_matrix()
    coords_r = coords[0, 0].reshape(-1)

    ftab, gtab = _build_tables(f1r, f2r, feat0r, w2, proj_w, pavg)
    out_flat = _sc_sample(ftab.reshape(-1), gtab.reshape(-1), coords_r)

    A = out_flat.reshape(H, 8, G, 16, L, 2, 9)
    out = A.transpose(4, 5, 2, 6, 0, 1, 3).reshape(2 * L * G * 9, H, W)
    return out[None].astype(jnp.float32)


# speedup vs baseline: 9.4225x; 2.8569x over previous
import functools

import jax
import jax.numpy as jnp
import numpy as np
from jax import lax
from jax.experimental import pallas as pl
from jax.experimental.pallas import tpu as pltpu
from jax.experimental.pallas import tpu_sc as plsc

G = 8
L = 4
R = 4
H = 64
W = 128
C = 64
COLS = 256
LEV_OFF = (0, 128, 192, 224)
LEV_D = (128, 64, 32, 16)
SLAB = G * 16 * COLS
OSLAB = G * 16 * 2 * L * 9
N_UNITS = H * (W // 16)
INV_SQRT_G = 1.0 / np.sqrt(float(G))


def _avg_matrix():
    P = np.zeros((128, 240), dtype=np.float32)
    col = 0
    for i in range(4):
        d = 128 >> i
        s = 1 << i
        for e in range(d):
            P[e * s:(e + 1) * s, col + e] = 1.0 / s
        col += d
    return P


def _table_matrices():
    P = _avg_matrix()
    Paug = np.concatenate([P, np.zeros((128, 16), np.float32)], axis=1)
    pf = Paug * INV_SQRT_G
    q = np.zeros((3, 128, 256), np.float32)
    for kd in range(3):
        Sh = np.zeros((128, 128), np.float32)
        for w2 in range(128):
            s = w2 + kd - 1
            if 0 <= s < 128:
                Sh[s, w2] = 1.0
        q[kd] = Sh @ Paug
    return jnp.asarray(pf), jnp.asarray(q)


def _tc_tables_body(f1_ref, f2_ref, feat0_ref, w72_ref, proj_ref, pf_ref,
                    q_ref, ftab_ref, gtab_ref):
    h = pl.program_id(0)
    cvs = []
    for ky in range(3):
        hh = h + ky - 1
        hc = jnp.clip(hh, 0, H - 1)
        a = f1_ref[hc]
        b = f2_ref[hc]
        if ky != 1:
            scale = jnp.where((hh >= 0) & (hh < H), 1.0, 0.0).astype(jnp.float32)
            a = a * scale
        row = []
        for g in range(G):
            cvg = lax.dot_general(a[g], b[g], (((0,), (0,)), ((), ())),
                                  preferred_element_type=jnp.float32)
            row.append(cvg)
        cvs.append(row)

    zr = jnp.zeros((1, 128), jnp.float32)
    slabs = []
    for ky in range(3):
        for g in range(G):
            s = cvs[ky][g]
            slabs.append(jnp.concatenate([zr, s[:-1]], axis=0))
            slabs.append(s)
            slabs.append(jnp.concatenate([s[1:], zr], axis=0))
    X72 = jnp.stack(slabs).astype(jnp.bfloat16)

    Z = lax.dot_general(w72_ref[...], X72, (((0,), (0,)), ((), ())),
                        preferred_element_type=jnp.float32)

    pbT = lax.dot_general(feat0_ref[h], proj_ref[...], (((0,), (1,)), ((), ())),
                          preferred_element_type=jnp.float32)

    for g in range(G):
        Fall = lax.dot_general(cvs[1][g], pf_ref[...], (((1,), (0,)), ((), ())),
                               preferred_element_type=jnp.float32)
        ftab_ref[0, :, g] = Fall.reshape(8, 16, COLS)
        acc = jnp.broadcast_to(pbT[:, g:g + 1], (128, COLS))
        for kd in range(3):
            acc = acc + lax.dot_general(Z[g * 3 + kd], q_ref[kd],
                                        (((1,), (0,)), ((), ())),
                                        preferred_element_type=jnp.float32)
        gtab_ref[0, :, g] = acc.reshape(8, 16, COLS)


def _build_tables(f1r, f2r, feat0r, w72, proj_w, pf, q):
    full = lambda s: pl.BlockSpec(s, lambda h: (0,) * len(s))
    return pl.pallas_call(
        _tc_tables_body,
        grid=(H,),
        in_specs=[
            full((H, G, 8, W)),
            full((H, G, 8, W)),
            full((H, C, W)),
            full((72, 24)),
            full((G, C)),
            full((128, COLS)),
            full((3, 128, COLS)),
        ],
        out_specs=[
            pl.BlockSpec((1, 8, G, 16, COLS), lambda h: (h, 0, 0, 0, 0)),
            pl.BlockSpec((1, 8, G, 16, COLS), lambda h: (h, 0, 0, 0, 0)),
        ],
        out_shape=[
            jax.ShapeDtypeStruct((H, 8, G, 16, COLS), jnp.float32),
            jax.ShapeDtypeStruct((H, 8, G, 16, COLS), jnp.float32),
        ],
        compiler_params=pltpu.CompilerParams(
            dimension_semantics=("arbitrary",),
        ),
    )(f1r, f2r, feat0r, w72, proj_w, pf, q)


def _sc_sample_body(ftab, gtab, coords_r, out_hbm,
                    abuf, bbuf, obuf, cbuf, i0buf, i1buf, wbuf, sema, semb):
    nc = 2
    wid = lax.axis_index("s") * nc + lax.axis_index("c")
    units_per = N_UNITS // 32
    lane = lax.iota(jnp.int32, 16)

    pltpu.sync_copy(coords_r.at[pl.ds(wid * units_per * 16, units_per * 16)],
                    cbuf)
    u0 = wid * units_per
    pltpu.async_copy(ftab.at[u0 // 8, u0 % 8], abuf, sema)

    def sample_pass(src_ref, pass_t):
        def g_body(g, _):
            gsplat = jnp.full((16,), 0, jnp.int32) + g
            for t in range(L * 9):
                col0 = (2 * (t // 9) + pass_t) * 9 + (t % 9)
                i0 = i0buf[pl.ds(t * 16, 16)]
                i1 = i1buf[pl.ds(t * 16, 16)]
                wf = wbuf[pl.ds(t * 16, 16)]
                v0 = plsc.load_gather(src_ref, [gsplat, lane, i0])
                v1 = plsc.load_gather(src_ref, [gsplat, lane, i1])
                obuf[pl.ds(col0 * 128 + g * 16, 16)] = v0 + (v1 - v0) * wf
            return 0

        lax.fori_loop(0, G, g_body, 0)

    def unit_body(j, _):
        u = wid * units_per + j
        c = cbuf[pl.ds(j * 16, 16)]
        for i in range(L):
            ci = c * (0.5 ** i)
            di = LEV_D[i]
            off = LEV_OFF[i]
            for k in range(9):
                pos = ci + float(k - R)
                t0 = pos.astype(jnp.int32)
                fl = jnp.where(t0.astype(jnp.float32) > pos, t0 - 1, t0)
                wfrac = pos - fl.astype(jnp.float32)
                x0 = jnp.clip(fl, 0, di - 1) + off
                x1 = jnp.clip(fl + 1, 0, di - 1) + off
                t = i * 9 + k
                i0buf[pl.ds(t * 16, 16)] = x0
                i1buf[pl.ds(t * 16, 16)] = x1
                wbuf[pl.ds(t * 16, 16)] = wfrac

        pltpu.make_async_copy(ftab.at[u // 8, u % 8], abuf, sema).wait()
        pltpu.async_copy(gtab.at[u // 8, u % 8], bbuf, semb)
        sample_pass(abuf, 0)
        pltpu.make_async_copy(gtab.at[u // 8, u % 8], bbuf, semb).wait()

        @pl.when(j < units_per - 1)
        def _():
            pltpu.async_copy(ftab.at[(u + 1) // 8, (u + 1) % 8], abuf, sema)

        sample_pass(bbuf, 1)
        pltpu.sync_copy(obuf, out_hbm.at[pl.ds(u * OSLAB, OSLAB)])
        return 0

    lax.fori_loop(0, units_per, unit_body, 0)


def _sc_sample(ftab, gtab, coords_flat):
    mesh = plsc.VectorSubcoreMesh(core_axis_name="c", subcore_axis_name="s")
    kern = functools.partial(
        pl.kernel,
        out_type=jax.ShapeDtypeStruct((N_UNITS * OSLAB,), jnp.float32),
        mesh=mesh,
        scratch_types=[
            pltpu.VMEM((G, 16, COLS), jnp.float32),
            pltpu.VMEM((G, 16, COLS), jnp.float32),
            pltpu.VMEM((OSLAB,), jnp.float32),
            pltpu.VMEM((N_UNITS // 32 * 16,), jnp.float32),
            pltpu.VMEM((L * 9 * 16,), jnp.int32),
            pltpu.VMEM((L * 9 * 16,), jnp.int32),
            pltpu.VMEM((L * 9 * 16,), jnp.float32),
            pltpu.SemaphoreType.DMA,
            pltpu.SemaphoreType.DMA,
        ],
        compiler_params=pltpu.CompilerParams(needs_layout_passes=False),
    )(_sc_sample_body)
    return kern(ftab, gtab, coords_flat)


def kernel(fmap1, fmap2, feat0, conv_w, proj_w, coords):
    f1r = fmap1[0].transpose(1, 0, 2).reshape(H, G, 8, W)
    f2r = fmap2[0].transpose(1, 0, 2).reshape(H, G, 8, W)
    feat0r = feat0[0].transpose(1, 0, 2)
    w72 = (conv_w.transpose(3, 1, 4, 0, 2).reshape(72, 24)
           * INV_SQRT_G).astype(jnp.bfloat16)
    pf, q = _table_matrices()
    coords_r = coords[0, 0].reshape(-1)

    ftab, gtab = _build_tables(f1r, f2r, feat0r, w72, proj_w, pf, q)
    out_flat = _sc_sample(ftab, gtab, coords_r)

    A = out_flat.reshape(H, 8, L, 2, 9, G, 16)
    out = A.transpose(2, 3, 5, 4, 0, 1, 6).reshape(2 * L * G * 9, H, W)
    return out[None].astype(jnp.float32)


# speedup vs baseline: 11.9034x; 1.2633x over previous
import functools

import jax
import jax.numpy as jnp
import numpy as np
from jax import lax
from jax.experimental import pallas as pl
from jax.experimental.pallas import tpu as pltpu
from jax.experimental.pallas import tpu_sc as plsc

G = 8
L = 4
R = 4
H = 64
W = 128
C = 64
COLS = 256
LEV_OFF = (0, 128, 192, 224)
LEV_D = (128, 64, 32, 16)
SLAB = G * 16 * COLS
OSLAB = G * 16 * 2 * L * 9
N_UNITS = H * (W // 16)
INV_SQRT_G = 1.0 / np.sqrt(float(G))


def _avg_matrix():
    P = np.zeros((128, 240), dtype=np.float32)
    col = 0
    for i in range(4):
        d = 128 >> i
        s = 1 << i
        for e in range(d):
            P[e * s:(e + 1) * s, col + e] = 1.0 / s
        col += d
    return P


def _table_matrices():
    P = _avg_matrix()
    Paug = np.concatenate([P, np.zeros((128, 16), np.float32)], axis=1)
    pf = Paug * INV_SQRT_G
    q = np.zeros((3, 128, 256), np.float32)
    for kd in range(3):
        Sh = np.zeros((128, 128), np.float32)
        for w2 in range(128):
            s = w2 + kd - 1
            if 0 <= s < 128:
                Sh[s, w2] = 1.0
        q[kd] = Sh @ Paug
    return jnp.asarray(pf), jnp.asarray(q)


def _tc_tables_body(f1_ref, f2_ref, feat0_ref, w72_ref, proj_ref, pf_ref,
                    q_ref, ftab_ref, gtab_ref):
    h = pl.program_id(0)
    cvs = []
    for ky in range(3):
        hh = h + ky - 1
        hc = jnp.clip(hh, 0, H - 1)
        a = f1_ref[hc]
        b = f2_ref[hc]
        if ky != 1:
            scale = jnp.where((hh >= 0) & (hh < H), 1.0, 0.0).astype(jnp.float32)
            a = a * scale
        row = []
        for g in range(G):
            cvg = lax.dot_general(a[g], b[g], (((0,), (0,)), ((), ())),
                                  preferred_element_type=jnp.float32)
            row.append(cvg)
        cvs.append(row)

    zr = jnp.zeros((1, 128), jnp.float32)
    slabs = []
    for ky in range(3):
        for g in range(G):
            s = cvs[ky][g]
            slabs.append(jnp.concatenate([zr, s[:-1]], axis=0))
            slabs.append(s)
            slabs.append(jnp.concatenate([s[1:], zr], axis=0))
    X72 = jnp.stack(slabs).astype(jnp.bfloat16)

    Z = lax.dot_general(w72_ref[...], X72, (((0,), (0,)), ((), ())),
                        preferred_element_type=jnp.float32)

    pbT = lax.dot_general(feat0_ref[h], proj_ref[...], (((0,), (1,)), ((), ())),
                          preferred_element_type=jnp.float32)

    for g in range(G):
        Fall = lax.dot_general(cvs[1][g], pf_ref[...], (((1,), (0,)), ((), ())),
                               preferred_element_type=jnp.float32)
        ftab_ref[0, :, g] = Fall.reshape(8, 16, COLS)
        acc = jnp.broadcast_to(pbT[:, g:g + 1], (128, COLS))
        for kd in range(3):
            acc = acc + lax.dot_general(Z[g * 3 + kd], q_ref[kd],
                                        (((1,), (0,)), ((), ())),
                                        preferred_element_type=jnp.float32)
        gtab_ref[0, :, g] = acc.reshape(8, 16, COLS)


def _build_tables(f1r, f2r, feat0r, w72, proj_w, pf, q):
    full = lambda s: pl.BlockSpec(s, lambda h: (0,) * len(s))
    return pl.pallas_call(
        _tc_tables_body,
        grid=(H,),
        in_specs=[
            full((H, G, 8, W)),
            full((H, G, 8, W)),
            full((H, C, W)),
            full((72, 24)),
            full((G, C)),
            full((128, COLS)),
            full((3, 128, COLS)),
        ],
        out_specs=[
            pl.BlockSpec((1, 8, G, 16, COLS), lambda h: (h, 0, 0, 0, 0)),
            pl.BlockSpec((1, 8, G, 16, COLS), lambda h: (h, 0, 0, 0, 0)),
        ],
        out_shape=[
            jax.ShapeDtypeStruct((H, 8, G, 16, COLS), jnp.float32),
            jax.ShapeDtypeStruct((H, 8, G, 16, COLS), jnp.float32),
        ],
        compiler_params=pltpu.CompilerParams(
            dimension_semantics=("arbitrary",),
        ),
    )(f1r, f2r, feat0r, w72, proj_w, pf, q)


def _sc_sample_body(ftab, gtab, coords_r, out_hbm,
                    abuf, bbuf, obuf, cbuf, i0buf, wbuf, sema, semb):
    nc = 2
    wid = lax.axis_index("s") * nc + lax.axis_index("c")
    units_per = N_UNITS // 32
    lane = lax.iota(jnp.int32, 16)

    pltpu.sync_copy(coords_r.at[pl.ds(wid * units_per * 16, units_per * 16)],
                    cbuf)
    u0 = wid * units_per
    pltpu.async_copy(ftab.at[u0 // 8, u0 % 8], abuf, sema)

    def sample_pass(src_ref, pass_t):
        def g_body(g, _):
            gsplat = jnp.full((16,), 0, jnp.int32) + g
            for i in range(L):
                wf = wbuf[pl.ds(i * 16, 16)]
                win = []
                for jw in range(10):
                    idx = i0buf[pl.ds((i * 10 + jw) * 16, 16)]
                    win.append(plsc.load_gather(src_ref, [gsplat, lane, idx]))
                for k in range(9):
                    col0 = (2 * i + pass_t) * 9 + k
                    v = win[k] + (win[k + 1] - win[k]) * wf
                    obuf[pl.ds(col0 * 128 + g * 16, 16)] = v
            return 0

        lax.fori_loop(0, G, g_body, 0)

    def unit_body(j, _):
        u = wid * units_per + j
        c = cbuf[pl.ds(j * 16, 16)]
        for i in range(L):
            ci = c * (0.5 ** i)
            di = LEV_D[i]
            off = LEV_OFF[i]
            t0 = ci.astype(jnp.int32)
            fl = jnp.where(t0.astype(jnp.float32) > ci, t0 - 1, t0)
            wbuf[pl.ds(i * 16, 16)] = ci - fl.astype(jnp.float32)
            for jw in range(10):
                i0buf[pl.ds((i * 10 + jw) * 16, 16)] = (
                    jnp.clip(fl + (jw - R), 0, di - 1) + off)

        pltpu.make_async_copy(ftab.at[u // 8, u % 8], abuf, sema).wait()
        pltpu.async_copy(gtab.at[u // 8, u % 8], bbuf, semb)
        sample_pass(abuf, 0)
        pltpu.make_async_copy(gtab.at[u // 8, u % 8], bbuf, semb).wait()

        @pl.when(j < units_per - 1)
        def _():
            pltpu.async_copy(ftab.at[(u + 1) // 8, (u + 1) % 8], abuf, sema)

        sample_pass(bbuf, 1)
        pltpu.sync_copy(obuf, out_hbm.at[pl.ds(u * OSLAB, OSLAB)])
        return 0

    lax.fori_loop(0, units_per, unit_body, 0)


def _sc_sample(ftab, gtab, coords_flat):
    mesh = plsc.VectorSubcoreMesh(core_axis_name="c", subcore_axis_name="s")
    kern = functools.partial(
        pl.kernel,
        out_type=jax.ShapeDtypeStruct((N_UNITS * OSLAB,), jnp.float32),
        mesh=mesh,
        scratch_types=[
            pltpu.VMEM((G, 16, COLS), jnp.float32),
            pltpu.VMEM((G, 16, COLS), jnp.float32),
            pltpu.VMEM((OSLAB,), jnp.float32),
            pltpu.VMEM((N_UNITS // 32 * 16,), jnp.float32),
            pltpu.VMEM((L * 10 * 16,), jnp.int32),
            pltpu.VMEM((L * 16,), jnp.float32),
            pltpu.SemaphoreType.DMA,
            pltpu.SemaphoreType.DMA,
        ],
        compiler_params=pltpu.CompilerParams(needs_layout_passes=False),
    )(_sc_sample_body)
    return kern(ftab, gtab, coords_flat)


def kernel(fmap1, fmap2, feat0, conv_w, proj_w, coords):
    f1r = fmap1[0].transpose(1, 0, 2).reshape(H, G, 8, W)
    f2r = fmap2[0].transpose(1, 0, 2).reshape(H, G, 8, W)
    feat0r = feat0[0].transpose(1, 0, 2)
    w72 = (conv_w.transpose(3, 1, 4, 0, 2).reshape(72, 24)
           * INV_SQRT_G).astype(jnp.bfloat16)
    pf, q = _table_matrices()
    coords_r = coords[0, 0].reshape(-1)

    ftab, gtab = _build_tables(f1r, f2r, feat0r, w72, proj_w, pf, q)
    out_flat = _sc_sample(ftab, gtab, coords_r)

    A = out_flat.reshape(H, 8, L, 2, 9, G, 16)
    out = A.transpose(2, 3, 5, 4, 0, 1, 6).reshape(2 * L * G * 9, H, W)
    return out[None].astype(jnp.float32)
